# baseline (speedup 1.0000x reference)
import jax
import jax.numpy as jnp
from jax.experimental import pallas as pl
from jax.experimental.pallas import tpu as pltpu


def _body(x_ref, emb_ref, wt_ref, we_ref, wgb_ref, bt_ref, bg_ref, be_ref,
          scal_ref, out_ref):
    emb = emb_ref[0]
    nrm2 = jnp.sum(emb * emb, axis=1, keepdims=True)
    en = emb * jax.lax.rsqrt(jnp.maximum(nrm2, 1e-24))
    en_bf = en.astype(jnp.bfloat16)

    gb = jax.lax.dot_general(
        en_bf, wgb_ref[...],
        dimension_numbers=(((1,), (1,)), ((), ())),
        preferred_element_type=jnp.float32)
    gate = gb[:, :8] + bg_ref[...]
    gw = jax.nn.softmax(gate, axis=1)
    iota = jax.lax.broadcasted_iota(jnp.int32, gw.shape, 1)
    i1 = jnp.argmax(gw, axis=1)[:, None]
    v1 = jnp.max(gw, axis=1, keepdims=True)
    gw2 = jnp.where(iota == i1, -1.0, gw)
    i2 = jnp.argmax(gw2, axis=1)[:, None]
    v2 = jnp.max(gw2, axis=1, keepdims=True)
    wmask = jnp.where(iota == i1, v1, 0.0) + jnp.where(iota == i2, v2, 0.0)

    tok = jax.lax.dot_general(
        en_bf, wt_ref[...],
        dimension_numbers=(((1,), (1,)), ((), ())),
        preferred_element_type=jnp.float32) + bt_ref[...]

    allexp = jax.lax.dot_general(
        en_bf, we_ref[...],
        dimension_numbers=(((1,), (2,)), ((), ())),
        preferred_element_type=jnp.float32)
    for e in range(8):
        tok = tok + wmask[:, e:e + 1] * allexp[:, e, :]
    tok = tok + jax.lax.dot_general(
        wmask, be_ref[...],
        dimension_numbers=(((1,), (0,)), ((), ())),
        preferred_element_type=jnp.float32)

    half_inv = scal_ref[0]
    bias_tok = gb[:, 8:9] + scal_ref[1]

    logit = jax.lax.dot_general(
        x_ref[0].astype(jnp.bfloat16), (tok * half_inv).astype(jnp.bfloat16),
        dimension_numbers=(((1,), (1,)), ((), ())),
        preferred_element_type=jnp.float32,
    ) + bias_tok.T
    out_ref[0] = jnp.clip(logit, -50000.0, 50000.0)


def kernel(x, embedding, Wt, bt, Wg, bg, We, be, bias_lang, bias0, log_scale):
    B, A, DO = x.shape
    L = embedding.shape[1]
    DL = embedding.shape[2]
    E = Wg.shape[0]

    wgb = jnp.concatenate(
        [Wg, bias_lang[None, :], jnp.zeros((7, DL), Wg.dtype)], axis=0
    ).astype(jnp.bfloat16)
    scal = jnp.stack([0.5 * jnp.exp(-log_scale[0]), bias0[0]])

    return pl.pallas_call(
        _body,
        grid=(B,),
        in_specs=[
            pl.BlockSpec((1, A, DO), lambda b: (b, 0, 0)),
            pl.BlockSpec((1, L, DL), lambda b: (b, 0, 0)),
            pl.BlockSpec((DO, DL), lambda b: (0, 0)),
            pl.BlockSpec((E, DO, DL), lambda b: (0, 0, 0)),
            pl.BlockSpec((2 * E, DL), lambda b: (0, 0)),
            pl.BlockSpec((DO,), lambda b: (0,)),
            pl.BlockSpec((E,), lambda b: (0,)),
            pl.BlockSpec((E, DO), lambda b: (0, 0)),
            pl.BlockSpec(memory_space=pltpu.SMEM),
        ],
        out_specs=pl.BlockSpec((1, A, L), lambda b: (b, 0, 0)),
        out_shape=jax.ShapeDtypeStruct((B, A, L), jnp.float32),
    )(x, embedding, Wt.astype(jnp.bfloat16), We.astype(jnp.bfloat16), wgb,
      bt, bg, be, scal)


# speedup vs baseline: 1.1355x; 1.1355x over previous
import jax
import jax.numpy as jnp
from jax.experimental import pallas as pl
from jax.experimental.pallas import tpu as pltpu

_DO = 256
_E = 8
_WCAT = _DO + _E * _DO + 128
_GCOL = _DO + _E * _DO
_BCOL = _GCOL + _E


def _body(x_ref, emb_ref, wcat_ref, bvec_ref, out_ref):
    emb = emb_ref[0]
    nrm2 = jnp.sum(emb * emb, axis=1, keepdims=True)
    en_bf = (emb * jax.lax.rsqrt(jnp.maximum(nrm2, 1e-24))).astype(jnp.bfloat16)

    y = jax.lax.dot_general(
        en_bf, wcat_ref[...],
        dimension_numbers=(((1,), (0,)), ((), ())),
        preferred_element_type=jnp.float32,
    ) + bvec_ref[...]

    gate = y[:, _GCOL:_GCOL + _E]
    gw = jax.nn.softmax(gate, axis=1)
    iota = jax.lax.broadcasted_iota(jnp.int32, gw.shape, 1)
    i1 = jnp.argmax(gw, axis=1)[:, None]
    v1 = jnp.max(gw, axis=1, keepdims=True)
    gw2 = jnp.where(iota == i1, -1.0, gw)
    i2 = jnp.argmax(gw2, axis=1)[:, None]
    v2 = jnp.max(gw2, axis=1, keepdims=True)
    wmask = (jnp.where(iota == i1, v1, 0.0)
             + jnp.where(iota == i2, v2, 0.0)).astype(jnp.bfloat16)

    ybf = y[:, _DO:_DO + _E * _DO].astype(jnp.bfloat16)
    tok = y[:, :_DO]
    for e in range(_E):
        tok = tok + (wmask[:, e:e + 1] * ybf[:, e * _DO:(e + 1) * _DO])

    bias_tok = y[:, _BCOL:_BCOL + 1]

    logit = jax.lax.dot_general(
        x_ref[0].astype(jnp.bfloat16), tok.astype(jnp.bfloat16),
        dimension_numbers=(((1,), (1,)), ((), ())),
        preferred_element_type=jnp.float32,
    ) + bias_tok.T
    out_ref[0] = jnp.clip(logit, -50000.0, 50000.0)


def kernel(x, embedding, Wt, bt, Wg, bg, We, be, bias_lang, bias0, log_scale):
    B, A, DO = x.shape
    L = embedding.shape[1]
    DL = embedding.shape[2]
    E = Wg.shape[0]

    inv = jnp.exp(-log_scale[0])
    half_inv = 0.5 * inv

    wcat = jnp.zeros((DL, _WCAT), dtype=jnp.float32)
    wcat = wcat.at[:, :DO].set(Wt.T * half_inv)
    wcat = wcat.at[:, DO:DO + E * DO].set(
        jnp.transpose(We, (2, 0, 1)).reshape(DL, E * DO) * half_inv)
    wcat = wcat.at[:, _GCOL:_GCOL + E].set(Wg.T)
    wcat = wcat.at[:, _BCOL].set(bias_lang)
    wcat = wcat.astype(jnp.bfloat16)

    bvec = jnp.zeros((1, _WCAT), dtype=jnp.float32)
    bvec = bvec.at[0, :DO].set(bt * half_inv)
    bvec = bvec.at[0, DO:DO + E * DO].set(be.reshape(E * DO) * half_inv)
    bvec = bvec.at[0, _GCOL:_GCOL + E].set(bg)
    bvec = bvec.at[0, _BCOL].set(bias0[0])

    return pl.pallas_call(
        _body,
        grid=(B,),
        in_specs=[
            pl.BlockSpec((1, A, DO), lambda b: (b, 0, 0)),
            pl.BlockSpec((1, L, DL), lambda b: (b, 0, 0)),
            pl.BlockSpec((DL, _WCAT), lambda b: (0, 0)),
            pl.BlockSpec((1, _WCAT), lambda b: (0, 0)),
        ],
        out_specs=pl.BlockSpec((1, A, L), lambda b: (b, 0, 0)),
        out_shape=jax.ShapeDtypeStruct((B, A, L), jnp.float32),
        compiler_params=pltpu.CompilerParams(
            dimension_semantics=("parallel",)),
    )(x, embedding, wcat, bvec)


# speedup vs baseline: 1.1879x; 1.0461x over previous
import jax
import jax.numpy as jnp
from jax.experimental import pallas as pl
from jax.experimental.pallas import tpu as pltpu

_DO = 256
_E = 8
_WCAT = _DO + _E * _DO + 128
_GCOL = _DO + _E * _DO
_BCOL = _GCOL + _E


def _body(x_ref, emb_ref, wcat_ref, bvec_ref, scal_ref, out_ref):
    emb = emb_ref[0]
    nrm2 = jnp.sum(emb * emb, axis=1, keepdims=True)
    en_bf = (emb * jax.lax.rsqrt(jnp.maximum(nrm2, 1e-24))).astype(jnp.bfloat16)

    y = jax.lax.dot_general(
        en_bf, wcat_ref[...],
        dimension_numbers=(((1,), (1,)), ((), ())),
        preferred_element_type=jnp.float32,
    ) + bvec_ref[...]

    gate = y[:, _GCOL:_GCOL + _E]
    gw = jax.nn.softmax(gate, axis=1)
    iota = jax.lax.broadcasted_iota(jnp.int32, gw.shape, 1)
    i1 = jnp.argmax(gw, axis=1)[:, None]
    v1 = jnp.max(gw, axis=1, keepdims=True)
    gw2 = jnp.where(iota == i1, -1.0, gw)
    i2 = jnp.argmax(gw2, axis=1)[:, None]
    v2 = jnp.max(gw2, axis=1, keepdims=True)
    wmask = (jnp.where(iota == i1, v1, 0.0)
             + jnp.where(iota == i2, v2, 0.0)).astype(jnp.bfloat16)

    ybf = y[:, :_GCOL].astype(jnp.bfloat16)
    tok = ybf[:, :_DO]
    for e in range(_E):
        lo = _DO + e * _DO
        tok = tok + wmask[:, e:e + 1] * ybf[:, lo:lo + _DO]

    half_inv = scal_ref[0]
    bias_tok = y[:, _BCOL:_BCOL + 1] + scal_ref[1]

    logit = jax.lax.dot_general(
        x_ref[0].astype(jnp.bfloat16),
        tok * jnp.bfloat16(half_inv),
        dimension_numbers=(((1,), (1,)), ((), ())),
        preferred_element_type=jnp.float32,
    ) + bias_tok.T
    out_ref[0] = jnp.clip(logit, -50000.0, 50000.0)


def kernel(x, embedding, Wt, bt, Wg, bg, We, be, bias_lang, bias0, log_scale):
    B, A, DO = x.shape
    L = embedding.shape[1]
    DL = embedding.shape[2]
    E = Wg.shape[0]

    wcat = jnp.concatenate(
        [Wt, We.reshape(E * DO, DL), Wg, bias_lang[None, :],
         jnp.zeros((_WCAT - _BCOL - 1, DL), jnp.float32)], axis=0,
    ).astype(jnp.bfloat16)

    bvec = jnp.concatenate(
        [bt, be.reshape(E * DO), bg, jnp.zeros((_WCAT - _BCOL,), jnp.float32)]
    )[None, :]

    scal = jnp.stack([0.5 * jnp.exp(-log_scale[0]), bias0[0]])

    return pl.pallas_call(
        _body,
        grid=(B,),
        in_specs=[
            pl.BlockSpec((1, A, DO), lambda b: (b, 0, 0)),
            pl.BlockSpec((1, L, DL), lambda b: (b, 0, 0)),
            pl.BlockSpec((_WCAT, DL), lambda b: (0, 0)),
            pl.BlockSpec((1, _WCAT), lambda b: (0, 0)),
            pl.BlockSpec(memory_space=pltpu.SMEM),
        ],
        out_specs=pl.BlockSpec((1, A, L), lambda b: (b, 0, 0)),
        out_shape=jax.ShapeDtypeStruct((B, A, L), jnp.float32),
        compiler_params=pltpu.CompilerParams(
            dimension_semantics=("parallel",)),
    )(x, embedding, wcat, bvec, scal)


# speedup vs baseline: 1.1935x; 1.0047x over previous
import jax
import jax.numpy as jnp
from jax.experimental import pallas as pl
from jax.experimental.pallas import tpu as pltpu

_DO = 256
_E = 8
_WCAT = _DO + _E * _DO + 128
_GCOL = _DO + _E * _DO
_BCOL = _GCOL + _E


def _body(x_ref, emb_ref, wcat_ref, bt_ref, bg_ref, be_ref, scal_ref, out_ref):
    emb = emb_ref[0]
    nrm2 = jnp.sum(emb * emb, axis=1, keepdims=True)
    en_bf = (emb * jax.lax.rsqrt(jnp.maximum(nrm2, 1e-24))).astype(jnp.bfloat16)

    y = jax.lax.dot_general(
        en_bf, wcat_ref[...],
        dimension_numbers=(((1,), (1,)), ((), ())),
        preferred_element_type=jnp.float32)

    gate = y[:, _GCOL:_GCOL + _E] + bg_ref[...]
    gw = jax.nn.softmax(gate, axis=1)
    iota = jax.lax.broadcasted_iota(jnp.int32, gw.shape, 1)
    i1 = jnp.argmax(gw, axis=1)[:, None]
    v1 = jnp.max(gw, axis=1, keepdims=True)
    gw2 = jnp.where(iota == i1, -1.0, gw)
    i2 = jnp.argmax(gw2, axis=1)[:, None]
    v2 = jnp.max(gw2, axis=1, keepdims=True)
    wmask = jnp.where(iota == i1, v1, 0.0) + jnp.where(iota == i2, v2, 0.0)
    wmask_bf = wmask.astype(jnp.bfloat16)

    ybf = y[:, :_GCOL].astype(jnp.bfloat16)
    tok = ybf[:, :_DO] + bt_ref[...].astype(jnp.bfloat16)
    for e in range(_E):
        lo = _DO + e * _DO
        tok = tok + wmask_bf[:, e:e + 1] * ybf[:, lo:lo + _DO]
    tok = tok + jax.lax.dot_general(
        wmask_bf, be_ref[...].astype(jnp.bfloat16),
        dimension_numbers=(((1,), (0,)), ((), ())),
        preferred_element_type=jnp.float32).astype(jnp.bfloat16)

    half_inv = 0.5 * jnp.exp(-scal_ref[0])
    bias_tok = y[:, _BCOL:_BCOL + 1] + scal_ref[1]

    logit = jax.lax.dot_general(
        x_ref[0].astype(jnp.bfloat16),
        tok * half_inv.astype(jnp.bfloat16),
        dimension_numbers=(((1,), (1,)), ((), ())),
        preferred_element_type=jnp.float32,
    ) + bias_tok.T
    out_ref[0] = jnp.clip(logit, -50000.0, 50000.0)


def kernel(x, embedding, Wt, bt, Wg, bg, We, be, bias_lang, bias0, log_scale):
    B, A, DO = x.shape
    L = embedding.shape[1]
    DL = embedding.shape[2]
    E = Wg.shape[0]

    wcat = jnp.concatenate(
        [Wt, We.reshape(E * DO, DL), Wg, bias_lang[None, :],
         jnp.zeros((_WCAT - _BCOL - 1, DL), jnp.float32)], axis=0,
    ).astype(jnp.bfloat16)

    scal = jnp.concatenate([log_scale, bias0])

    return pl.pallas_call(
        _body,
        grid=(B,),
        in_specs=[
            pl.BlockSpec((1, A, DO), lambda b: (b, 0, 0)),
            pl.BlockSpec((1, L, DL), lambda b: (b, 0, 0)),
            pl.BlockSpec((_WCAT, DL), lambda b: (0, 0)),
            pl.BlockSpec((DO,), lambda b: (0,)),
            pl.BlockSpec((E,), lambda b: (0,)),
            pl.BlockSpec((E, DO), lambda b: (0, 0)),
            pl.BlockSpec(memory_space=pltpu.SMEM),
        ],
        out_specs=pl.BlockSpec((1, A, L), lambda b: (b, 0, 0)),
        out_shape=jax.ShapeDtypeStruct((B, A, L), jnp.float32),
        compiler_params=pltpu.CompilerParams(
            dimension_semantics=("parallel",)),
    )(x, embedding, wcat, bt, bg, be, scal)
